# baseline (speedup 1.0000x reference)
import functools

import jax
import jax.numpy as jnp
from jax import lax
from jax.experimental import pallas as pl
from jax.experimental.pallas import tpu as pltpu
from jax.experimental.pallas import tpu_sc as plsc

N = 10000
E = 320000
D = 128
H = 256
G = 16

NC = 2
NS = 16
NW = NC * NS
CH = 88
NK = 120
NSG = 4
NCHUNKP = NW * NK
EPAD = NCHUNKP * CH
NP = 10240
RPS = NP // NS

@functools.lru_cache(maxsize=None)
def _build_segment_sum_sc():
    mesh = plsc.VectorSubcoreMesh(
        core_axis_name="c", subcore_axis_name="s",
        num_cores=NC, num_subcores=NS)

    @functools.partial(
        pl.kernel,
        out_type=jax.ShapeDtypeStruct((NC, NP, D), jnp.float32),
        mesh=mesh,
        scratch_types=[
            [pltpu.VMEM((2 * NSG, CH), jnp.int32) for _ in range(2)],
            [pltpu.VMEM((CH, D), jnp.float32) for _ in range(4)],
            pltpu.VMEM_SHARED((NP, D), jnp.float32),
            [pltpu.SemaphoreType.DMA for _ in range(4)],
            [pltpu.SemaphoreType.DMA for _ in range(2)],
            [pltpu.SemaphoreType.DMA for _ in range(4)],
        ],
    )
    def _segment_sum_sc(x_hbm, idx_hbm, zero_hbm, out_hbm,
                        idxb, rows, acc_sh, gs, isem, ss):
        c = lax.axis_index("c")
        s = lax.axis_index("s")
        w = s * NC + c

        pltpu.sync_copy(zero_hbm.at[pl.ds(s * RPS, RPS)],
                        acc_sh.at[pl.ds(s * RPS, RPS)])
        plsc.subcore_barrier()

        gbase = w * (NK // NSG)

        def load_idx(sg_rel, ib):
            pltpu.async_copy(idx_hbm.at[gbase + sg_rel], idxb[ib], isem[ib])

        def wait_idx(ib):
            pltpu.make_async_copy(idx_hbm.at[gbase], idxb[ib],
                                  isem[ib]).wait()

        def gather(ib, r, b):
            pltpu.async_copy(x_hbm.at[idxb[ib].at[r]], rows[b], gs[b])

        def wait_gather(b):
            pltpu.make_async_copy(x_hbm.at[idxb[0].at[0]], rows[b],
                                  gs[b]).wait()

        def scatter(ib, r, b):
            pltpu.async_copy(rows[b], acc_sh.at[idxb[ib].at[NSG + r]],
                             ss[b], add=True)

        def wait_scatter(b):
            pltpu.make_async_copy(rows[b], acc_sh.at[idxb[0].at[NSG]],
                                  ss[b]).wait()

        pltpu.sync_copy(idx_hbm.at[gbase], idxb[0])
        gather(0, 0, 0)
        gather(0, 1, 1)

        npair = NK // (2 * NSG)

        def jbody(j, carry):
            for q in range(8):
                b = q % 4
                ib = q // 4
                r = q % 4
                wait_gather(b)
                scatter(ib, r, b)
                b2 = (q + 2) % 4
                nq = q + 2
                if q == 2:
                    wait_idx(1)
                if nq < 8:
                    if q < 2:
                        @pl.when(j > 0)
                        def _():
                            wait_scatter(b2)
                    else:
                        wait_scatter(b2)
                    gather(nq // 4, nq % 4, b2)
                else:
                    @pl.when(j + 1 < npair)
                    def _():
                        if nq == 8:
                            wait_idx(0)
                        wait_scatter(b2)
                        gather(0, nq - 8, b2)
                if q == 1:
                    load_idx(2 * j + 1, 1)
                if q == 5:
                    @pl.when(j + 1 < npair)
                    def _():
                        load_idx(2 * j + 2, 0)
            return carry

        lax.fori_loop(0, npair, jbody, 0)
        for b in range(4):
            wait_scatter(b)

        plsc.subcore_barrier()
        pltpu.sync_copy(acc_sh.at[pl.ds(s * RPS, RPS)],
                        out_hbm.at[c, pl.ds(s * RPS, RPS)])

    return _segment_sum_sc


def _bn_cols(t, g, b):
    mu = jnp.mean(t, axis=0, keepdims=True)
    var = jnp.mean((t - mu) ** 2, axis=0, keepdims=True)
    return (t - mu) / jnp.sqrt(var + 1e-5) * g + b


def _gin_mid_body(eps_ref, x_ref, a0_ref, a1_ref, w1_ref, g1_ref, b1_ref,
                  w2_ref, g2_ref, b2_ref, mask_ref, out_ref):
    h = (1.0 + eps_ref[0, 0]) * x_ref[...] + a0_ref[:N] + a1_ref[:N]
    t = jnp.dot(h, w1_ref[...], preferred_element_type=jnp.float32)
    t = _bn_cols(t, g1_ref[...], b1_ref[...])
    s = jnp.maximum(t, 0.0)
    u = jnp.dot(s, w2_ref[...], preferred_element_type=jnp.float32)
    u = _bn_cols(u, g2_ref[...], b2_ref[...])
    out_ref[...] = jnp.maximum(u * mask_ref[...], 0.0)


def _gin_last_body(eps_ref, x_ref, a0_ref, a1_ref, w1_ref, g1_ref, b1_ref,
                   w2_ref, mask_ref, batch_ref, out_ref):
    h = (1.0 + eps_ref[0, 0]) * x_ref[...] + a0_ref[:N] + a1_ref[:N]
    t = jnp.dot(h, w1_ref[...], preferred_element_type=jnp.float32)
    t = _bn_cols(t, g1_ref[...], b1_ref[...])
    s = jnp.maximum(t, 0.0)
    u = jnp.dot(s, w2_ref[...], preferred_element_type=jnp.float32)
    xf = jnp.maximum(u * mask_ref[...], 0.0)
    oh = (lax.broadcasted_iota(jnp.int32, (G, N), 0)
          == batch_ref[...]).astype(jnp.float32)
    sums = jnp.dot(oh, xf, preferred_element_type=jnp.float32)
    cnt = jnp.sum(oh, axis=1, keepdims=True)
    out_ref[...] = sums / jnp.maximum(cnt, 1.0)


_tc_params = pltpu.CompilerParams(vmem_limit_bytes=100 * 1024 * 1024)

_mid_call = pl.pallas_call(
    _gin_mid_body,
    out_shape=jax.ShapeDtypeStruct((N, D), jnp.float32),
    in_specs=[pl.BlockSpec(memory_space=pltpu.SMEM)]
    + [pl.BlockSpec(memory_space=pltpu.VMEM)] * 10,
    out_specs=pl.BlockSpec(memory_space=pltpu.VMEM),
    compiler_params=_tc_params,
)

_last_call = pl.pallas_call(
    _gin_last_body,
    out_shape=jax.ShapeDtypeStruct((G, D), jnp.float32),
    in_specs=[pl.BlockSpec(memory_space=pltpu.SMEM)]
    + [pl.BlockSpec(memory_space=pltpu.VMEM)] * 9,
    out_specs=pl.BlockSpec(memory_space=pltpu.VMEM),
    compiler_params=_tc_params,
)


def _make_mask(i):
    od = D if i < 6 else 2
    dk = jax.random.key(42)
    keep = jax.random.bernoulli(jax.random.fold_in(dk, i), 0.8, (N, od))
    m = jnp.where(keep, jnp.float32(1.0) / jnp.float32(0.8), jnp.float32(0.0))
    if od != D:
        m = jnp.pad(m, ((0, 0), (0, D - od)))
    return m


def kernel(x, edge_index, batch, params):
    npad = EPAD - E
    pad_iota = jnp.arange(npad, dtype=jnp.int32)
    srcp = jnp.concatenate([edge_index[0], pad_iota % N])
    dstp = jnp.concatenate([edge_index[1], N + pad_iota % (NP - N)])
    idx = jnp.concatenate(
        [srcp.reshape(NCHUNKP // NSG, NSG, CH),
         dstp.reshape(NCHUNKP // NSG, NSG, CH)], axis=1)
    zeros = jnp.zeros((NP, D), jnp.float32)
    batch_row = batch.reshape(1, N)
    out = None
    seg_sum = _build_segment_sum_sc()
    for i, p in enumerate(params):
        parts = seg_sum(x, idx, zeros)
        eps2 = p['eps'].reshape(1, 1)
        if i < 6:
            x = _mid_call(eps2, x, parts[0], parts[1], p['W1'],
                          p['g1'].reshape(1, H), p['b1'].reshape(1, H),
                          p['W2'], p['g2'].reshape(1, D),
                          p['b2'].reshape(1, D), _make_mask(i))
        else:
            w2p = jnp.pad(p['W2'], ((0, 0), (0, D - 2)))
            out = _last_call(eps2, x, parts[0], parts[1], p['W1'],
                             p['g1'].reshape(1, H), p['b1'].reshape(1, H),
                             w2p, _make_mask(6), batch_row)
    return out[:, :2]


# speedup vs baseline: 1.0821x; 1.0821x over previous
import functools

import jax
import jax.numpy as jnp
from jax import lax
from jax.experimental import pallas as pl
from jax.experimental.pallas import tpu as pltpu
from jax.experimental.pallas import tpu_sc as plsc

N = 10000
E = 320000
D = 128
H = 256
G = 16

NC = 2
NS = 16
NW = NC * NS
CH = 128
NK = 80
NSG = 4
NCHUNKP = NW * NK
EPAD = NCHUNKP * CH
NP = 10240
RPS = NP // NS

@functools.lru_cache(maxsize=None)
def _build_segment_sum_sc():
    mesh = plsc.VectorSubcoreMesh(
        core_axis_name="c", subcore_axis_name="s",
        num_cores=NC, num_subcores=NS)

    @functools.partial(
        pl.kernel,
        out_type=jax.ShapeDtypeStruct((NC, NP, D), jnp.float32),
        mesh=mesh,
        scratch_types=[
            [pltpu.VMEM((2 * NSG, CH), jnp.int32) for _ in range(2)],
            [pltpu.VMEM((CH, D), jnp.float32) for _ in range(2)],
            pltpu.VMEM_SHARED((NP, D), jnp.float32),
            [pltpu.SemaphoreType.DMA for _ in range(2)],
            [pltpu.SemaphoreType.DMA for _ in range(2)],
        ],
    )
    def _segment_sum_sc(x_hbm, idx_hbm, zero_hbm, out_hbm,
                        idxb, rows, acc_sh, gs, isem):
        c = lax.axis_index("c")
        s = lax.axis_index("s")
        w = s * NC + c

        pltpu.sync_copy(zero_hbm.at[pl.ds(s * RPS, RPS)],
                        acc_sh.at[pl.ds(s * RPS, RPS)])
        plsc.subcore_barrier()

        gbase = w * (NK // NSG)

        def load_idx(sg_rel, ib):
            pltpu.async_copy(idx_hbm.at[gbase + sg_rel], idxb[ib], isem[ib])

        def wait_idx(ib):
            pltpu.make_async_copy(idx_hbm.at[gbase], idxb[ib],
                                  isem[ib]).wait()

        def gather(ib, r, b):
            pltpu.async_copy(x_hbm.at[idxb[ib].at[r]], rows[b], gs[b])

        def wait_gather(b):
            pltpu.make_async_copy(x_hbm.at[idxb[0].at[0]], rows[b],
                                  gs[b]).wait()

        def scatter(ib, r, b):
            pltpu.sync_copy(rows[b], acc_sh.at[idxb[ib].at[NSG + r]],
                            add=True)

        pltpu.sync_copy(idx_hbm.at[gbase], idxb[0])
        load_idx(1, 1)
        gather(0, 0, 0)
        gather(0, 1, 1)

        npair = NK // (2 * NSG)

        def jbody(j, carry):
            for q in range(8):
                b = q % 2
                ib = q // 4
                r = q % 4
                wait_gather(b)
                scatter(ib, r, b)
                if q == 2:
                    wait_idx(1)
                nq = q + 2
                if nq < 8:
                    gather(nq // 4, nq % 4, b)
                else:
                    @pl.when(j + 1 < npair)
                    def _():
                        if nq == 8:
                            wait_idx(0)
                        gather(0, nq - 8, b)
                if q == 3:

                    @pl.when(j + 1 < npair)
                    def _():
                        load_idx(2 * j + 2, 0)
                if q == 7:

                    @pl.when(j + 1 < npair)
                    def _():
                        load_idx(2 * j + 3, 1)
            return carry

        lax.fori_loop(0, npair, jbody, 0)

        plsc.subcore_barrier()
        pltpu.sync_copy(acc_sh.at[pl.ds(s * RPS, RPS)],
                        out_hbm.at[c, pl.ds(s * RPS, RPS)])

    return _segment_sum_sc


def _bn_cols(t, g, b):
    mu = jnp.mean(t, axis=0, keepdims=True)
    var = jnp.mean((t - mu) ** 2, axis=0, keepdims=True)
    return (t - mu) / jnp.sqrt(var + 1e-5) * g + b


def _gin_mid_body(eps_ref, x_ref, a0_ref, a1_ref, w1_ref, g1_ref, b1_ref,
                  w2_ref, g2_ref, b2_ref, mask_ref, out_ref):
    h = (1.0 + eps_ref[0, 0]) * x_ref[...] + a0_ref[:N] + a1_ref[:N]
    t = jnp.dot(h, w1_ref[...], preferred_element_type=jnp.float32)
    t = _bn_cols(t, g1_ref[...], b1_ref[...])
    s = jnp.maximum(t, 0.0)
    u = jnp.dot(s, w2_ref[...], preferred_element_type=jnp.float32)
    u = _bn_cols(u, g2_ref[...], b2_ref[...])
    out_ref[...] = jnp.maximum(u * mask_ref[...], 0.0)


def _gin_last_body(eps_ref, x_ref, a0_ref, a1_ref, w1_ref, g1_ref, b1_ref,
                   w2_ref, mask_ref, batch_ref, out_ref):
    h = (1.0 + eps_ref[0, 0]) * x_ref[...] + a0_ref[:N] + a1_ref[:N]
    t = jnp.dot(h, w1_ref[...], preferred_element_type=jnp.float32)
    t = _bn_cols(t, g1_ref[...], b1_ref[...])
    s = jnp.maximum(t, 0.0)
    u = jnp.dot(s, w2_ref[...], preferred_element_type=jnp.float32)
    xf = jnp.maximum(u * mask_ref[...], 0.0)
    oh = (lax.broadcasted_iota(jnp.int32, (G, N), 0)
          == batch_ref[...]).astype(jnp.float32)
    sums = jnp.dot(oh, xf, preferred_element_type=jnp.float32)
    cnt = jnp.sum(oh, axis=1, keepdims=True)
    out_ref[...] = sums / jnp.maximum(cnt, 1.0)


_tc_params = pltpu.CompilerParams(vmem_limit_bytes=100 * 1024 * 1024)

_mid_call = pl.pallas_call(
    _gin_mid_body,
    out_shape=jax.ShapeDtypeStruct((N, D), jnp.float32),
    in_specs=[pl.BlockSpec(memory_space=pltpu.SMEM)]
    + [pl.BlockSpec(memory_space=pltpu.VMEM)] * 10,
    out_specs=pl.BlockSpec(memory_space=pltpu.VMEM),
    compiler_params=_tc_params,
)

_last_call = pl.pallas_call(
    _gin_last_body,
    out_shape=jax.ShapeDtypeStruct((G, D), jnp.float32),
    in_specs=[pl.BlockSpec(memory_space=pltpu.SMEM)]
    + [pl.BlockSpec(memory_space=pltpu.VMEM)] * 9,
    out_specs=pl.BlockSpec(memory_space=pltpu.VMEM),
    compiler_params=_tc_params,
)


def _make_mask(i):
    od = D if i < 6 else 2
    dk = jax.random.key(42)
    keep = jax.random.bernoulli(jax.random.fold_in(dk, i), 0.8, (N, od))
    m = jnp.where(keep, jnp.float32(1.0) / jnp.float32(0.8), jnp.float32(0.0))
    if od != D:
        m = jnp.pad(m, ((0, 0), (0, D - od)))
    return m


def kernel(x, edge_index, batch, params):
    npad = EPAD - E
    pad_iota = jnp.arange(npad, dtype=jnp.int32)
    srcp = jnp.concatenate([edge_index[0], pad_iota % N])
    dstp = jnp.concatenate([edge_index[1], N + pad_iota % (NP - N)])
    idx = jnp.concatenate(
        [srcp.reshape(NCHUNKP // NSG, NSG, CH),
         dstp.reshape(NCHUNKP // NSG, NSG, CH)], axis=1)
    zeros = jnp.zeros((NP, D), jnp.float32)
    batch_row = batch.reshape(1, N)
    out = None
    seg_sum = _build_segment_sum_sc()
    for i, p in enumerate(params):
        parts = seg_sum(x, idx, zeros)
        eps2 = p['eps'].reshape(1, 1)
        if i < 6:
            x = _mid_call(eps2, x, parts[0], parts[1], p['W1'],
                          p['g1'].reshape(1, H), p['b1'].reshape(1, H),
                          p['W2'], p['g2'].reshape(1, D),
                          p['b2'].reshape(1, D), _make_mask(i))
        else:
            w2p = jnp.pad(p['W2'], ((0, 0), (0, D - 2)))
            out = _last_call(eps2, x, parts[0], parts[1], p['W1'],
                             p['g1'].reshape(1, H), p['b1'].reshape(1, H),
                             w2p, _make_mask(6), batch_row)
    return out[:, :2]
